# Initial kernel scaffold; baseline (speedup 1.0000x reference)
import jax
import jax.numpy as jnp
from jax.experimental import pallas as pl


def kernel(input, table, W1, b1, W2, b2):
    raise NotImplementedError("write your pallas kernel here")


# speedup vs baseline: 117.9624x; 117.9624x over previous
import functools

import jax
import jax.numpy as jnp
from jax import lax
from jax.experimental import pallas as pl
from jax.experimental.pallas import tpu as pltpu
from jax.experimental.pallas import tpu_sc as plsc

_VOCAB = 10000
_D = 64
_B = 16384
_H = 200
_NC = 2
_NS = 16
_NW = _NC * _NS
_RPT = _B // _NW


def _fold_body(table_ref, w1_ref, b1_ref, w2_ref, b2_ref, t_ref):
    w2 = w2_ref[...]
    v = lax.dot_general(w2, w1_ref[...], (((1,), (0,)), ((), ())),
                        preferred_element_type=jnp.float32)
    c = jnp.sum(w2[0:1, :] * b1_ref[...]) + b2_ref[0, 0]
    t = lax.dot_general(table_ref[...], v, (((1,), (1,)), ((), ())),
                        preferred_element_type=jnp.float32)
    t_ref[...] = (t[:, 0:1] + c) * (1.0 / _H)


_fold = pl.pallas_call(
    _fold_body,
    out_shape=jax.ShapeDtypeStruct((_VOCAB, 1), jnp.float32),
)


_sc_mesh = plsc.VectorSubcoreMesh(core_axis_name="c", subcore_axis_name="s")


@functools.partial(
    pl.kernel,
    out_type=jax.ShapeDtypeStruct((_B,), jnp.float32),
    mesh=_sc_mesh,
    scratch_types=[
        pltpu.VMEM((_RPT * _H,), jnp.int32),
        pltpu.VMEM((_VOCAB,), jnp.float32),
        pltpu.VMEM((_RPT,), jnp.float32),
    ],
    compiler_params=pltpu.CompilerParams(needs_layout_passes=False),
)
def _sc_sum(idx_hbm, t_hbm, out_hbm, idx_v, t_v, out_v):
    wid = lax.axis_index("s") * _NC + lax.axis_index("c")
    base = wid * _RPT
    pltpu.sync_copy(t_hbm, t_v)
    pltpu.sync_copy(idx_hbm.at[pl.ds(base * _H, _RPT * _H)], idx_v)
    lane = lax.iota(jnp.int32, 16)
    for j in range(_RPT // 16):
        rows = j * 16 + lane

        def body(l, acc):
            flat = rows * _H + l
            idxv = plsc.load_gather(idx_v, [flat])
            vals = plsc.load_gather(t_v, [idxv])
            return acc + vals

        acc = lax.fori_loop(0, _H, body, jnp.zeros((16,), jnp.float32))
        out_v[pl.ds(j * 16, 16)] = acc
    pltpu.sync_copy(out_v, out_hbm.at[pl.ds(base, _RPT)])


@jax.jit
def kernel(input, table, W1, b1, W2, b2):
    idx = input.astype(jnp.int32).reshape(-1)
    w2p = jnp.zeros((8, _D), jnp.float32).at[0].set(W2[0])
    t = _fold(table, W1, b1.reshape(1, _D), w2p, b2.reshape(1, 1))
    out = _sc_sum(idx, t.reshape(_VOCAB))
    return out.reshape(_B, 1)


# speedup vs baseline: 163.5260x; 1.3863x over previous
import functools

import jax
import jax.numpy as jnp
from jax import lax
from jax.experimental import pallas as pl
from jax.experimental.pallas import tpu as pltpu
from jax.experimental.pallas import tpu_sc as plsc

_VOCAB = 10000
_D = 64
_B = 16384
_H = 200
_NC = 2
_NS = 16
_NW = _NC * _NS
_RPT = _B // _NW
_U = 8


def _fold_body(table_ref, w1_ref, b1_ref, w2_ref, b2_ref, t_ref):
    w2 = w2_ref[...]
    v = lax.dot_general(w2, w1_ref[...], (((1,), (0,)), ((), ())),
                        preferred_element_type=jnp.float32)
    c = jnp.sum(w2[0:1, :] * b1_ref[...]) + b2_ref[0, 0]
    t = lax.dot_general(table_ref[...], v, (((1,), (1,)), ((), ())),
                        preferred_element_type=jnp.float32)
    t_ref[...] = (t[:, 0:1] + c) * (1.0 / _H)


_fold = pl.pallas_call(
    _fold_body,
    out_shape=jax.ShapeDtypeStruct((_VOCAB, 1), jnp.float32),
)


_sc_mesh = plsc.VectorSubcoreMesh(core_axis_name="c", subcore_axis_name="s")


@functools.partial(
    pl.kernel,
    out_type=jax.ShapeDtypeStruct((_B,), jnp.float32),
    mesh=_sc_mesh,
    scratch_types=[
        pltpu.VMEM((_RPT * _H,), jnp.int32),
        pltpu.VMEM((_VOCAB,), jnp.float32),
        pltpu.VMEM((_RPT,), jnp.float32),
    ],
    compiler_params=pltpu.CompilerParams(needs_layout_passes=False),
)
def _sc_sum(idx_hbm, t_hbm, out_hbm, idx_v, t_v, out_v):
    wid = lax.axis_index("s") * _NC + lax.axis_index("c")
    base = wid * _RPT
    pltpu.sync_copy(t_hbm, t_v)
    pltpu.sync_copy(idx_hbm.at[pl.ds(base * _H, _RPT * _H)], idx_v)
    lane = lax.iota(jnp.int32, 16)
    for j in range(_RPT // 16):
        rows = j * 16 + lane

        def body(i, accs):
            l = i * _U
            new = []
            for u in range(_U):
                flat = rows * _H + (l + u)
                idxv = plsc.load_gather(idx_v, [flat])
                vals = plsc.load_gather(t_v, [idxv])
                new.append(accs[u] + vals)
            return tuple(new)

        zero = jnp.zeros((16,), jnp.float32)
        accs = lax.fori_loop(0, _H // _U, body, (zero,) * _U)
        acc = accs[0]
        for u in range(1, _U):
            acc = acc + accs[u]
        out_v[pl.ds(j * 16, 16)] = acc
    pltpu.sync_copy(out_v, out_hbm.at[pl.ds(base, _RPT)])


@jax.jit
def kernel(input, table, W1, b1, W2, b2):
    idx = input.astype(jnp.int32).reshape(-1)
    w2p = jnp.zeros((8, _D), jnp.float32).at[0].set(W2[0])
    t = _fold(table, W1, b1.reshape(1, _D), w2p, b2.reshape(1, 1))
    out = _sc_sum(idx, t.reshape(_VOCAB))
    return out.reshape(_B, 1)


# speedup vs baseline: 163.6279x; 1.0006x over previous
import functools

import jax
import jax.numpy as jnp
from jax import lax
from jax.experimental import pallas as pl
from jax.experimental.pallas import tpu as pltpu
from jax.experimental.pallas import tpu_sc as plsc

_VOCAB = 10000
_D = 64
_B = 16384
_H = 200
_NC = 2
_NS = 16
_NW = _NC * _NS
_RPT = _B // _NW
_U = 8


def _fold_body(table_ref, w1_ref, b1_ref, w2_ref, b2_ref, t_ref):
    w2 = w2_ref[...]
    v = lax.dot_general(w2, w1_ref[...], (((1,), (0,)), ((), ())),
                        preferred_element_type=jnp.float32)
    c = jnp.sum(w2[0:1, :] * b1_ref[...]) + b2_ref[0, 0]
    t = lax.dot_general(table_ref[...], v, (((1,), (1,)), ((), ())),
                        preferred_element_type=jnp.float32)
    t_ref[...] = (t[:, 0:1] + c) * (1.0 / _H)


_fold = pl.pallas_call(
    _fold_body,
    out_shape=jax.ShapeDtypeStruct((_VOCAB, 1), jnp.float32),
)


_sc_mesh = plsc.VectorSubcoreMesh(core_axis_name="c", subcore_axis_name="s")


@functools.partial(
    pl.kernel,
    out_type=jax.ShapeDtypeStruct((_B,), jnp.float32),
    mesh=_sc_mesh,
    scratch_types=[
        pltpu.VMEM((_RPT * _H // 128, 128), jnp.int32),
        pltpu.VMEM((_VOCAB,), jnp.float32),
        pltpu.VMEM((_RPT,), jnp.float32),
    ],
    compiler_params=pltpu.CompilerParams(needs_layout_passes=False),
)
def _sc_sum(idx_hbm, t_hbm, out_hbm, idx_v, t_v, out_v):
    wid = lax.axis_index("s") * _NC + lax.axis_index("c")
    base = wid * _RPT
    pltpu.sync_copy(t_hbm, t_v)
    rows_128 = _RPT * _H // 128
    pltpu.sync_copy(idx_hbm.at[pl.ds(wid * rows_128, rows_128), :], idx_v)
    lane = lax.iota(jnp.int32, 16)
    for j in range(_RPT // 16):
        rows = j * 16 + lane

        def body(i, accs):
            l = i * _U
            new = []
            for u in range(_U):
                flat = rows * _H + (l + u)
                idxv = plsc.load_gather(idx_v, [flat >> 7, flat & 127])
                vals = plsc.load_gather(t_v, [idxv])
                new.append(accs[u] + vals)
            return tuple(new)

        zero = jnp.zeros((16,), jnp.float32)
        accs = lax.fori_loop(0, _H // _U, body, (zero,) * _U)
        acc = accs[0]
        for u in range(1, _U):
            acc = acc + accs[u]
        out_v[pl.ds(j * 16, 16)] = acc
    pltpu.sync_copy(out_v, out_hbm.at[pl.ds(base, _RPT)])


@jax.jit
def kernel(input, table, W1, b1, W2, b2):
    idx = input.astype(jnp.int32).reshape(_B * _H // 128, 128)
    w2p = jnp.zeros((8, _D), jnp.float32).at[0].set(W2[0])
    t = _fold(table, W1, b1.reshape(1, _D), w2p, b2.reshape(1, 1))
    out = _sc_sum(idx, t.reshape(_VOCAB))
    return out.reshape(_B, 1)


# speedup vs baseline: 259.8852x; 1.5883x over previous
import functools

import jax
import jax.numpy as jnp
from jax import lax
from jax.experimental import pallas as pl
from jax.experimental.pallas import tpu as pltpu
from jax.experimental.pallas import tpu_sc as plsc

_VOCAB = 10000
_D = 64
_B = 16384
_H = 200
_NC = 2
_NS = 16
_NW = _NC * _NS
_RPT = _B // _NW
_U = 8


def _fold_body(table_ref, w1_ref, b1_ref, w2_ref, b2_ref, t_ref):
    w2 = w2_ref[...]
    v = lax.dot_general(w2, w1_ref[...], (((1,), (0,)), ((), ())),
                        preferred_element_type=jnp.float32)
    c = jnp.sum(w2[0:1, :] * b1_ref[...]) + b2_ref[0, 0]
    t = lax.dot_general(table_ref[...], v, (((1,), (1,)), ((), ())),
                        preferred_element_type=jnp.float32)
    t_ref[...] = (t[:, 0:1] + c) * (1.0 / _H)


_fold = pl.pallas_call(
    _fold_body,
    out_shape=jax.ShapeDtypeStruct((_VOCAB, 1), jnp.float32),
)


_sc_mesh = plsc.VectorSubcoreMesh(core_axis_name="c", subcore_axis_name="s")


@functools.partial(
    pl.kernel,
    out_type=jax.ShapeDtypeStruct((_B,), jnp.float32),
    mesh=_sc_mesh,
    scratch_types=[
        pltpu.VMEM((_H, _RPT), jnp.int32),
        pltpu.VMEM((_VOCAB,), jnp.float32),
        pltpu.VMEM((_RPT,), jnp.float32),
    ],
    compiler_params=pltpu.CompilerParams(needs_layout_passes=False),
)
def _sc_sum(idx_hbm, t_hbm, out_hbm, idx_v, t_v, out_v):
    wid = lax.axis_index("s") * _NC + lax.axis_index("c")
    base = wid * _RPT
    pltpu.sync_copy(t_hbm, t_v)
    pltpu.sync_copy(idx_hbm.at[:, pl.ds(base, _RPT)], idx_v)
    for j in range(_RPT // 16):

        def body(i, accs):
            new = []
            for u in range(_U):
                idxv = idx_v[i * _U + u, pl.ds(j * 16, 16)]
                vals = plsc.load_gather(t_v, [idxv])
                new.append(accs[u] + vals)
            return tuple(new)

        zero = jnp.zeros((16,), jnp.float32)
        accs = lax.fori_loop(0, _H // _U, body, (zero,) * _U)
        acc = accs[0]
        for u in range(1, _U):
            acc = acc + accs[u]
        out_v[pl.ds(j * 16, 16)] = acc
    pltpu.sync_copy(out_v, out_hbm.at[pl.ds(base, _RPT)])


@jax.jit
def kernel(input, table, W1, b1, W2, b2):
    idx = input.astype(jnp.int32).T
    w2p = jnp.zeros((8, _D), jnp.float32).at[0].set(W2[0])
    t = _fold(table, W1, b1.reshape(1, _D), w2p, b2.reshape(1, 1))
    out = _sc_sum(idx, t.reshape(_VOCAB))
    return out.reshape(_B, 1)


# speedup vs baseline: 272.5496x; 1.0487x over previous
import functools

import jax
import jax.numpy as jnp
from jax import lax
from jax.experimental import pallas as pl
from jax.experimental.pallas import tpu as pltpu
from jax.experimental.pallas import tpu_sc as plsc

_VOCAB = 10000
_D = 64
_B = 16384
_H = 200
_NC = 2
_NS = 16
_NW = _NC * _NS
_RPT = _B // _NW
_U = 8
_HC0 = 104
_HC1 = _H - _HC0


def _fold_body(tablet_ref, w1_ref, b1_ref, w2_ref, b2_ref, t_ref):
    w2 = w2_ref[...]
    v = lax.dot_general(w2, w1_ref[...], (((1,), (0,)), ((), ())),
                        preferred_element_type=jnp.float32)
    c = jnp.sum(w2[0:1, :] * b1_ref[...]) + b2_ref[0, 0]
    t = lax.dot_general(tablet_ref[...], v, (((0,), (1,)), ((), ())),
                        preferred_element_type=jnp.float32)
    t_ref[...] = (t[:, 0:1] + c) * (1.0 / _H)


_fold = pl.pallas_call(
    _fold_body,
    out_shape=jax.ShapeDtypeStruct((_VOCAB, 1), jnp.float32),
)


_sc_mesh = plsc.VectorSubcoreMesh(core_axis_name="c", subcore_axis_name="s")


@functools.partial(
    pl.kernel,
    out_type=jax.ShapeDtypeStruct((_B,), jnp.float32),
    mesh=_sc_mesh,
    scratch_types=[
        pltpu.VMEM((_HC0, _RPT), jnp.int32),
        pltpu.VMEM((_HC1, _RPT), jnp.int32),
        pltpu.VMEM((_VOCAB,), jnp.float32),
        pltpu.VMEM((_RPT,), jnp.float32),
        pltpu.SemaphoreType.DMA,
        pltpu.SemaphoreType.DMA,
    ],
    compiler_params=pltpu.CompilerParams(needs_layout_passes=False),
)
def _sc_sum(idx_hbm, t_hbm, out_hbm, idx_v0, idx_v1, t_v, out_v, sem0, sem1):
    wid = lax.axis_index("s") * _NC + lax.axis_index("c")
    base = wid * _RPT
    cp0 = pltpu.async_copy(idx_hbm.at[pl.ds(0, _HC0), pl.ds(base, _RPT)], idx_v0, sem0)
    cp1 = pltpu.async_copy(idx_hbm.at[pl.ds(_HC0, _HC1), pl.ds(base, _RPT)], idx_v1, sem1)
    pltpu.sync_copy(t_hbm, t_v)
    zero = jnp.zeros((16,), jnp.float32)
    for half, (cp, idx_v, n_iter) in enumerate(((cp0, idx_v0, _HC0 // _U), (cp1, idx_v1, _HC1 // _U))):
        cp.wait()
        for j in range(_RPT // 16):

            def body(i, accs):
                new = []
                for u in range(_U):
                    idxv = idx_v[i * _U + u, pl.ds(j * 16, 16)]
                    vals = plsc.load_gather(t_v, [idxv])
                    new.append(accs[u] + vals)
                return tuple(new)

            accs = lax.fori_loop(0, n_iter, body, (zero,) * _U)
            acc = accs[0]
            for u in range(1, _U):
                acc = acc + accs[u]
            if half == 0:
                out_v[pl.ds(j * 16, 16)] = acc
            else:
                out_v[pl.ds(j * 16, 16)] = out_v[pl.ds(j * 16, 16)] + acc
    pltpu.sync_copy(out_v, out_hbm.at[pl.ds(base, _RPT)])


@jax.jit
def kernel(input, table, W1, b1, W2, b2):
    idx = input.astype(jnp.int32).T
    w2p = jnp.zeros((8, _D), jnp.float32).at[0].set(W2[0])
    t = _fold(table.T, W1, b1.reshape(1, _D), w2p, b2.reshape(1, 1))
    out = _sc_sum(idx, t.reshape(_VOCAB))
    return out.reshape(_B, 1)


# speedup vs baseline: 274.2040x; 1.0061x over previous
import functools

import jax
import jax.numpy as jnp
from jax import lax
from jax.experimental import pallas as pl
from jax.experimental.pallas import tpu as pltpu
from jax.experimental.pallas import tpu_sc as plsc

_VOCAB = 10000
_D = 64
_B = 16384
_H = 200
_NC = 2
_NS = 16
_NW = _NC * _NS
_RPT = _B // _NW
_U = 8
_HC0 = 104
_HC1 = _H - _HC0


def _fold_body(tablet_ref, w1_ref, b1_ref, w2_ref, b2_ref, t_ref):
    w2 = w2_ref[...]
    v = lax.dot_general(w2, w1_ref[...], (((1,), (0,)), ((), ())),
                        preferred_element_type=jnp.float32)
    c = jnp.sum(w2[0:1, :] * b1_ref[...]) + b2_ref[0, 0]
    t = lax.dot_general(tablet_ref[...], v, (((0,), (1,)), ((), ())),
                        preferred_element_type=jnp.float32)
    t_ref[...] = (t[:, 0] + c) * (1.0 / _H)


_fold = pl.pallas_call(
    _fold_body,
    out_shape=jax.ShapeDtypeStruct((_VOCAB,), jnp.float32),
)


_sc_mesh = plsc.VectorSubcoreMesh(core_axis_name="c", subcore_axis_name="s")


@functools.partial(
    pl.kernel,
    out_type=jax.ShapeDtypeStruct((_B,), jnp.float32),
    mesh=_sc_mesh,
    scratch_types=[
        pltpu.VMEM((_HC0, _RPT), jnp.int32),
        pltpu.VMEM((_HC1, _RPT), jnp.int32),
        pltpu.VMEM((_VOCAB,), jnp.float32),
        pltpu.VMEM((_RPT,), jnp.float32),
        pltpu.SemaphoreType.DMA,
        pltpu.SemaphoreType.DMA,
    ],
    compiler_params=pltpu.CompilerParams(needs_layout_passes=False),
)
def _sc_sum(idx_hbm, t_hbm, out_hbm, idx_v0, idx_v1, t_v, out_v, sem0, sem1):
    wid = lax.axis_index("s") * _NC + lax.axis_index("c")
    base = wid * _RPT
    cp0 = pltpu.async_copy(idx_hbm.at[pl.ds(0, _HC0), pl.ds(base, _RPT)], idx_v0, sem0)
    cp1 = pltpu.async_copy(idx_hbm.at[pl.ds(_HC0, _HC1), pl.ds(base, _RPT)], idx_v1, sem1)
    pltpu.sync_copy(t_hbm, t_v)
    zero = jnp.zeros((16,), jnp.float32)
    for half, (cp, idx_v, n_iter) in enumerate(((cp0, idx_v0, _HC0 // _U), (cp1, idx_v1, _HC1 // _U))):
        cp.wait()
        for j in range(_RPT // 16):

            def body(i, accs):
                new = []
                for u in range(_U):
                    idxv = idx_v[i * _U + u, pl.ds(j * 16, 16)]
                    vals = plsc.load_gather(t_v, [idxv])
                    new.append(accs[u] + vals)
                return tuple(new)

            accs = lax.fori_loop(0, n_iter, body, (zero,) * _U)
            acc = accs[0]
            for u in range(1, _U):
                acc = acc + accs[u]
            if half == 0:
                out_v[pl.ds(j * 16, 16)] = acc
            else:
                out_v[pl.ds(j * 16, 16)] = out_v[pl.ds(j * 16, 16)] + acc
    pltpu.sync_copy(out_v, out_hbm.at[pl.ds(base, _RPT)])


@jax.jit
def kernel(input, table, W1, b1, W2, b2):
    idx = input.astype(jnp.int32).T
    w2p = jnp.zeros((8, _D), jnp.float32).at[0].set(W2[0])
    t = _fold(table.T, W1, b1.reshape(1, _D), w2p, b2.reshape(1, 1))
    out = _sc_sum(idx, t)
    return out.reshape(_B, 1)


# speedup vs baseline: 316.8442x; 1.1555x over previous
import functools

import jax
import jax.numpy as jnp
from jax import lax
from jax.experimental import pallas as pl
from jax.experimental.pallas import tpu as pltpu
from jax.experimental.pallas import tpu_sc as plsc

_VOCAB = 10000
_D = 64
_B = 16384
_H = 200
_NC = 2
_NS = 16
_NW = _NC * _NS
_RPT = _B // _NW
_U = 8
_HC0 = 104
_HC1 = _H - _HC0


def _fold_body(tablet_ref, w1_ref, b1_ref, w2_ref, b2_ref, t_ref):
    w2 = w2_ref[...]
    v = lax.dot_general(w2, w1_ref[...], (((1,), (0,)), ((), ())),
                        preferred_element_type=jnp.float32)
    c = jnp.sum(w2[0:1, :] * b1_ref[...]) + b2_ref[0, 0]
    t = lax.dot_general(v, tablet_ref[...], (((1,), (0,)), ((), ())),
                        preferred_element_type=jnp.float32)
    t_ref[...] = (t[0, :] + c) * (1.0 / _H)


_fold = pl.pallas_call(
    _fold_body,
    out_shape=jax.ShapeDtypeStruct((_VOCAB,), jnp.float32),
)


_sc_mesh = plsc.VectorSubcoreMesh(core_axis_name="c", subcore_axis_name="s")


@functools.partial(
    pl.kernel,
    out_type=jax.ShapeDtypeStruct((_B,), jnp.float32),
    mesh=_sc_mesh,
    scratch_types=[
        pltpu.VMEM((_HC0, _RPT), jnp.int32),
        pltpu.VMEM((_HC1, _RPT), jnp.int32),
        pltpu.VMEM((_VOCAB,), jnp.float32),
        pltpu.VMEM((_RPT,), jnp.float32),
        pltpu.SemaphoreType.DMA,
        pltpu.SemaphoreType.DMA,
    ],
    compiler_params=pltpu.CompilerParams(needs_layout_passes=False),
)
def _sc_sum(idx_hbm, t_hbm, out_hbm, idx_v0, idx_v1, t_v, out_v, sem0, sem1):
    wid = lax.axis_index("s") * _NC + lax.axis_index("c")
    base = wid * _RPT
    cp0 = pltpu.async_copy(idx_hbm.at[pl.ds(0, _HC0), pl.ds(base, _RPT)], idx_v0, sem0)
    cp1 = pltpu.async_copy(idx_hbm.at[pl.ds(_HC0, _HC1), pl.ds(base, _RPT)], idx_v1, sem1)
    pltpu.sync_copy(t_hbm, t_v)
    zero = jnp.zeros((16,), jnp.float32)
    for half, (cp, idx_v, n_iter) in enumerate(((cp0, idx_v0, _HC0 // _U), (cp1, idx_v1, _HC1 // _U))):
        cp.wait()
        for j in range(_RPT // 16):

            def body(i, accs):
                new = []
                for u in range(_U):
                    idxv = idx_v[i * _U + u, pl.ds(j * 16, 16)]
                    vals = plsc.load_gather(t_v, [idxv])
                    new.append(accs[u] + vals)
                return tuple(new)

            accs = lax.fori_loop(0, n_iter, body, (zero,) * _U)
            acc = accs[0]
            for u in range(1, _U):
                acc = acc + accs[u]
            if half == 0:
                out_v[pl.ds(j * 16, 16)] = acc
            else:
                out_v[pl.ds(j * 16, 16)] = out_v[pl.ds(j * 16, 16)] + acc
    pltpu.sync_copy(out_v, out_hbm.at[pl.ds(base, _RPT)])


@jax.jit
def kernel(input, table, W1, b1, W2, b2):
    idx = input.astype(jnp.int32).T
    w2p = jnp.zeros((8, _D), jnp.float32).at[0].set(W2[0])
    t = _fold(table.T, W1, b1.reshape(1, _D), w2p, b2.reshape(1, 1))
    out = _sc_sum(idx, t)
    return out.reshape(_B, 1)
